# Initial kernel scaffold; baseline (speedup 1.0000x reference)
import jax
import jax.numpy as jnp
from jax.experimental import pallas as pl


def kernel(agent_lca, lca_len, emb_table, proj_w):
    raise NotImplementedError("write your pallas kernel here")


# speedup vs baseline: 18.9488x; 18.9488x over previous
import functools

import jax
import jax.numpy as jnp
from jax import lax
from jax.experimental import pallas as pl
from jax.experimental.pallas import tpu as pltpu
from jax.experimental.pallas import tpu_sc as plsc

N = 128
L = 8
H = 8
D = 32
ROWS = 323
P = N * N
NW = 32
PPW = P // NW
EPW = PPW * L
PB = 16
NPB = PPW // PB


def _project_kernel(emb_ref, wmat_ref, o_ref):
    o_ref[...] = jnp.dot(
        emb_ref[...],
        wmat_ref[...],
        preferred_element_type=jnp.float32,
        precision=lax.Precision.HIGHEST,
    )


def _project(emb_table, proj_w):
    row = jnp.arange(H * D, dtype=jnp.int32)
    col = jnp.arange(H, dtype=jnp.int32)
    wmat = jnp.where(
        col[None, :] == row[:, None] // D,
        jnp.tile(proj_w[0], H)[:, None],
        0.0,
    )
    return pl.pallas_call(
        _project_kernel,
        out_shape=jax.ShapeDtypeStruct((ROWS, H), jnp.float32),
    )(emb_table, wmat)


def _sc_body(lca_hbm, len_hbm, t_hbm, out_hbm, idx_v, len_v, t_v, out_v):
    wid = lax.axis_index("s") * 2 + lax.axis_index("c")
    base_p = wid * PPW
    base_e = wid * EPW

    pltpu.sync_copy(t_hbm, t_v)
    pltpu.sync_copy(lca_hbm.at[pl.ds(base_e, EPW)], idx_v)
    pltpu.sync_copy(len_hbm.at[pl.ds(base_p, PPW)], len_v)

    lanes = lax.iota(jnp.int32, 16)
    lanes8 = lanes * L
    hcasts = [jnp.full((16,), h, jnp.int32) for h in range(H)]

    def block(pb, _):
        p0 = pb * PB
        lenf = len_v[pl.ds(p0, 16)].astype(jnp.float32)
        invl = 1.0 / lenf
        e0 = pb * (PB * L)
        accs = [jnp.zeros((16,), jnp.float32) for _ in range(H)]
        for l in range(L):
            gi = plsc.load_gather(idx_v, [e0 + l + lanes8])
            for h in range(H):
                accs[h] = accs[h] + plsc.load_gather(t_v, [gi, hcasts[h]])
        for h in range(H):
            out_v[h, pl.ds(p0, 16)] = accs[h] * invl
        return _

    lax.fori_loop(0, NPB, block, None)

    for h in range(H):
        pltpu.sync_copy(out_v.at[h], out_hbm.at[h, pl.ds(base_p, PPW)])


@functools.partial(jax.jit, static_argnums=())
def _sc_gather(lca_flat, len_flat, t):
    mesh = plsc.VectorSubcoreMesh(core_axis_name="c", subcore_axis_name="s")
    f = functools.partial(
        pl.kernel,
        out_type=jax.ShapeDtypeStruct((H, P), jnp.float32),
        mesh=mesh,
        scratch_types=[
            pltpu.VMEM((EPW,), jnp.int32),
            pltpu.VMEM((PPW,), jnp.int32),
            pltpu.VMEM((ROWS, H), jnp.float32),
            pltpu.VMEM((H, PPW), jnp.float32),
        ],
        compiler_params=pltpu.CompilerParams(needs_layout_passes=False),
    )(_sc_body)
    return f(lca_flat, len_flat, t)


def kernel(agent_lca, lca_len, emb_table, proj_w):
    lca_flat = agent_lca.reshape(-1).astype(jnp.int32)
    len_flat = lca_len.reshape(-1).astype(jnp.int32)
    t = _project(emb_table, proj_w)
    out = _sc_gather(lca_flat, len_flat, t)
    return out.reshape(H, N, N)


# speedup vs baseline: 19.4296x; 1.0254x over previous
import functools

import jax
import jax.numpy as jnp
from jax import lax
from jax.experimental import pallas as pl
from jax.experimental.pallas import tpu as pltpu
from jax.experimental.pallas import tpu_sc as plsc

N = 128
L = 8
H = 8
D = 32
ROWS = 323
P = N * N
NW = 32
PPW = P // NW
EPW = PPW * L
PB = 16
NPB = PPW // PB


def _project_kernel(emb_ref, wmat_ref, o_ref):
    o_ref[...] = jnp.dot(
        emb_ref[...],
        wmat_ref[...],
        preferred_element_type=jnp.float32,
        precision=lax.Precision.HIGHEST,
    )


def _project(emb_table, proj_w):
    row = jnp.arange(H * D, dtype=jnp.int32)
    col = jnp.arange(H, dtype=jnp.int32)
    wmat = jnp.where(
        col[None, :] == row[:, None] // D,
        jnp.tile(proj_w[0], H)[:, None],
        0.0,
    )
    return pl.pallas_call(
        _project_kernel,
        out_shape=jax.ShapeDtypeStruct((ROWS, H), jnp.float32),
    )(emb_table, wmat)


def _sc_body(lca_hbm, len_hbm, t_hbm, out_hbm, idx_v, len_v, t_v, out_v,
             sem_in, sem_out):
    wid = lax.axis_index("s") * 2 + lax.axis_index("c")
    base_p = wid * PPW
    base_e = wid * EPW

    d1 = pltpu.async_copy(t_hbm, t_v, sem_in)
    d2 = pltpu.async_copy(lca_hbm.at[pl.ds(base_e, EPW)], idx_v, sem_in)
    d3 = pltpu.async_copy(len_hbm.at[pl.ds(base_p, PPW)], len_v, sem_in)
    d1.wait()
    d2.wait()
    d3.wait()

    lanes = lax.iota(jnp.int32, 16)
    lanes8 = lanes * L
    hcasts = [jnp.full((16,), h, jnp.int32) for h in range(H)]

    def block(pb, _):
        p0 = pb * PB
        lenf = len_v[pl.ds(p0, 16)].astype(jnp.float32)
        invl = 1.0 / lenf
        e0 = pb * (PB * L)
        accs = [jnp.zeros((16,), jnp.float32) for _ in range(H)]
        for l in range(L):
            gi = plsc.load_gather(idx_v, [e0 + l + lanes8])
            for h in range(H):
                accs[h] = accs[h] + plsc.load_gather(t_v, [gi, hcasts[h]])
        for h in range(H):
            out_v[h, pl.ds(p0, 16)] = accs[h] * invl
        return _

    lax.fori_loop(0, NPB, block, None)

    outs = [
        pltpu.async_copy(out_v.at[h], out_hbm.at[h, pl.ds(base_p, PPW)], sem_out)
        for h in range(H)
    ]
    for d in outs:
        d.wait()


@functools.partial(jax.jit, static_argnums=())
def _sc_gather(lca_flat, len_flat, t):
    mesh = plsc.VectorSubcoreMesh(core_axis_name="c", subcore_axis_name="s")
    f = functools.partial(
        pl.kernel,
        out_type=jax.ShapeDtypeStruct((H, P), jnp.float32),
        mesh=mesh,
        scratch_types=[
            pltpu.VMEM((EPW,), jnp.int32),
            pltpu.VMEM((PPW,), jnp.int32),
            pltpu.VMEM((ROWS, H), jnp.float32),
            pltpu.VMEM((H, PPW), jnp.float32),
            pltpu.SemaphoreType.DMA,
            pltpu.SemaphoreType.DMA,
        ],
        compiler_params=pltpu.CompilerParams(needs_layout_passes=False),
    )(_sc_body)
    return f(lca_flat, len_flat, t)


def kernel(agent_lca, lca_len, emb_table, proj_w):
    lca_flat = agent_lca.reshape(-1).astype(jnp.int32)
    len_flat = lca_len.reshape(-1).astype(jnp.int32)
    t = _project(emb_table, proj_w)
    out = _sc_gather(lca_flat, len_flat, t)
    return out.reshape(H, N, N)


# speedup vs baseline: 24.7173x; 1.2721x over previous
import functools

import jax
import jax.numpy as jnp
from jax import lax
from jax.experimental import pallas as pl
from jax.experimental.pallas import tpu as pltpu
from jax.experimental.pallas import tpu_sc as plsc

N = 128
L = 8
H = 8
D = 32
TW = 9
ROWS = 323
P = N * N
NW = 32
PPW = P // NW
PB = 16
NPB = PPW // PB


def _project_kernel(emb_ref, wmat_ref, o_ref):
    o_ref[...] = jnp.dot(
        emb_ref[...],
        wmat_ref[...],
        preferred_element_type=jnp.float32,
        precision=lax.Precision.HIGHEST,
    )


def _project(emb_table, proj_w):
    row = jnp.arange(H * D, dtype=jnp.int32)
    col = jnp.arange(TW, dtype=jnp.int32)
    wmat = jnp.where(
        col[None, :] == row[:, None] // D,
        jnp.tile(proj_w[0], H)[:, None],
        0.0,
    )
    return pl.pallas_call(
        _project_kernel,
        out_shape=jax.ShapeDtypeStruct((ROWS, TW), jnp.float32),
    )(emb_table, wmat)


def _sc_body(lca_hbm, len_hbm, t_hbm, out_hbm, idx_v, len_v, t_v, out_v,
             sem_in, sem_out):
    wid = lax.axis_index("s") * 2 + lax.axis_index("c")
    base_p = wid * PPW

    ins = [pltpu.async_copy(t_hbm, t_v, sem_in),
           pltpu.async_copy(len_hbm.at[pl.ds(base_p, PPW)], len_v, sem_in)]
    ins += [
        pltpu.async_copy(lca_hbm.at[l, pl.ds(base_p, PPW)], idx_v.at[l], sem_in)
        for l in range(L)
    ]
    for d in ins:
        d.wait()

    hcasts = [jnp.full((16,), h, jnp.int32) for h in range(H)]

    def block(pb, _):
        p0 = pb * PB
        lenf = len_v[pl.ds(p0, 16)].astype(jnp.float32)
        invl = 1.0 / lenf
        accs = [jnp.zeros((16,), jnp.float32) for _ in range(H)]
        for l in range(L):
            gi = idx_v[l, pl.ds(p0, 16)]
            for h in range(H):
                accs[h] = accs[h] + plsc.load_gather(t_v, [gi, hcasts[h]])
        for h in range(H):
            out_v[h, pl.ds(p0, 16)] = accs[h] * invl
        return _

    lax.fori_loop(0, NPB, block, None)

    outs = [
        pltpu.async_copy(out_v.at[h], out_hbm.at[h, pl.ds(base_p, PPW)], sem_out)
        for h in range(H)
    ]
    for d in outs:
        d.wait()


@functools.partial(jax.jit, static_argnums=())
def _sc_gather(lca_t, len_flat, t):
    mesh = plsc.VectorSubcoreMesh(core_axis_name="c", subcore_axis_name="s")
    f = functools.partial(
        pl.kernel,
        out_type=jax.ShapeDtypeStruct((H, P), jnp.float32),
        mesh=mesh,
        scratch_types=[
            pltpu.VMEM((L, PPW), jnp.int32),
            pltpu.VMEM((PPW,), jnp.int32),
            pltpu.VMEM((ROWS, TW), jnp.float32),
            pltpu.VMEM((H, PPW), jnp.float32),
            pltpu.SemaphoreType.DMA,
            pltpu.SemaphoreType.DMA,
        ],
        compiler_params=pltpu.CompilerParams(needs_layout_passes=False),
    )(_sc_body)
    return f(lca_t, len_flat, t)


def kernel(agent_lca, lca_len, emb_table, proj_w):
    lca_t = agent_lca.reshape(P, L).astype(jnp.int32).T
    len_flat = lca_len.reshape(-1).astype(jnp.int32)
    t = _project(emb_table, proj_w)
    out = _sc_gather(lca_t, len_flat, t)
    return out.reshape(H, N, N)


# speedup vs baseline: 37.8917x; 1.5330x over previous
import functools

import jax
import jax.numpy as jnp
from jax import lax
from jax.experimental import pallas as pl
from jax.experimental.pallas import tpu as pltpu
from jax.experimental.pallas import tpu_sc as plsc

N = 128
L = 8
H = 8
D = 32
TW = 9
ROWS = 323
P = N * N
NW = 32
PPW = P // NW
PB = 16
NPB = PPW // PB


def _project_kernel(emb_ref, wmat_ref, o_ref):
    o_ref[...] = jnp.dot(
        emb_ref[...],
        wmat_ref[...],
        preferred_element_type=jnp.float32,
        precision=lax.Precision.HIGHEST,
    )


def _project(emb_table, proj_w):
    row = jnp.arange(H * D, dtype=jnp.int32)
    col = jnp.arange(TW, dtype=jnp.int32)
    wmat = jnp.where(
        col[None, :] == row[:, None] // D,
        jnp.tile(proj_w[0], H)[:, None],
        0.0,
    )
    return pl.pallas_call(
        _project_kernel,
        out_shape=jax.ShapeDtypeStruct((ROWS, TW), jnp.float32),
    )(emb_table, wmat)


def _sc_body(lca_hbm, len_hbm, t_hbm, out_hbm, idx_v, len_v, t_v, out_v,
             sem_in, sem_out):
    wid = lax.axis_index("s") * 2 + lax.axis_index("c")
    base_p = wid * PPW

    ins = [pltpu.async_copy(t_hbm, t_v, sem_in),
           pltpu.async_copy(len_hbm.at[pl.ds(base_p, PPW)], len_v, sem_in)]
    ins += [
        pltpu.async_copy(lca_hbm.at[l, pl.ds(base_p, PPW)], idx_v.at[l], sem_in)
        for l in range(L)
    ]
    for d in ins:
        d.wait()

    def block(pb, _):
        p0 = pb * PB
        lenf = len_v[pl.ds(p0, 16)].astype(jnp.float32)
        invl = 1.0 / lenf
        accs = [jnp.zeros((16,), jnp.float32) for _ in range(H)]
        for l in range(L):
            gi = idx_v[l, pl.ds(p0, 16)] * TW
            for h in range(H):
                accs[h] = accs[h] + plsc.load_gather(t_v, [gi + h])
        for h in range(H):
            out_v[h, pl.ds(p0, 16)] = accs[h] * invl
        return _

    lax.fori_loop(0, NPB, block, None)

    outs = [
        pltpu.async_copy(out_v.at[h], out_hbm.at[h, pl.ds(base_p, PPW)], sem_out)
        for h in range(H)
    ]
    for d in outs:
        d.wait()


@functools.partial(jax.jit, static_argnums=())
def _sc_gather(lca_t, len_flat, t):
    mesh = plsc.VectorSubcoreMesh(core_axis_name="c", subcore_axis_name="s")
    f = functools.partial(
        pl.kernel,
        out_type=jax.ShapeDtypeStruct((H, P), jnp.float32),
        mesh=mesh,
        scratch_types=[
            pltpu.VMEM((L, PPW), jnp.int32),
            pltpu.VMEM((PPW,), jnp.int32),
            pltpu.VMEM((ROWS * TW,), jnp.float32),
            pltpu.VMEM((H, PPW), jnp.float32),
            pltpu.SemaphoreType.DMA,
            pltpu.SemaphoreType.DMA,
        ],
        compiler_params=pltpu.CompilerParams(needs_layout_passes=False),
    )(_sc_body)
    return f(lca_t, len_flat, t)


def kernel(agent_lca, lca_len, emb_table, proj_w):
    lca_t = agent_lca.reshape(P, L).astype(jnp.int32).T
    len_flat = lca_len.reshape(-1).astype(jnp.int32)
    t = _project(emb_table, proj_w).reshape(ROWS * TW)
    out = _sc_gather(lca_t, len_flat, t)
    return out.reshape(H, N, N)


# speedup vs baseline: 44.0509x; 1.1625x over previous
import functools

import jax
import jax.numpy as jnp
from jax import lax
from jax.experimental import pallas as pl
from jax.experimental.pallas import tpu as pltpu
from jax.experimental.pallas import tpu_sc as plsc

N = 128
L = 8
H = 8
D = 32
TW = 9
ROWS = 323
P = N * N
NW = 32
PPW = P // NW
RPW = PPW // N
PB = 16
NPB = PPW // PB


def _project_kernel(emb_ref, w_ref, o_ref):
    w = w_ref[0, :]
    w_tiled = jnp.concatenate([w] * H)
    row = lax.broadcasted_iota(jnp.int32, (H * D, TW), 0)
    col = lax.broadcasted_iota(jnp.int32, (H * D, TW), 1)
    wmat = jnp.where(col == row // D, w_tiled[:, None], 0.0)
    o_ref[...] = lax.dot_general(
        wmat,
        emb_ref[...],
        (((0,), (1,)), ((), ())),
        preferred_element_type=jnp.float32,
        precision=lax.Precision.HIGHEST,
    )


def _project(emb_table, proj_w):
    return pl.pallas_call(
        _project_kernel,
        out_shape=jax.ShapeDtypeStruct((TW, ROWS), jnp.float32),
    )(emb_table, proj_w)


def _sc_body(lca_hbm, len_hbm, t_hbm, out_hbm, idx_v, len_v, t_v, out_v,
             sem_in, sem_out):
    wid = lax.axis_index("s") * 2 + lax.axis_index("c")
    base_p = wid * PPW

    ins = [pltpu.async_copy(t_hbm, t_v, sem_in),
           pltpu.async_copy(len_hbm.at[pl.ds(base_p, PPW)], len_v, sem_in)]
    ins += [
        pltpu.async_copy(lca_hbm.at[l, pl.ds(base_p, PPW)], idx_v.at[l], sem_in)
        for l in range(L)
    ]
    for d in ins:
        d.wait()

    hcasts = [jnp.full((16,), h, jnp.int32) for h in range(H)]

    def block(pb, _):
        p0 = pb * PB
        row = p0 // N
        col = p0 % N
        lenf = len_v[pl.ds(p0, 16)].astype(jnp.float32)
        invl = 1.0 / lenf
        accs = [jnp.zeros((16,), jnp.float32) for _ in range(H)]
        for l in range(L):
            gi = idx_v[l, pl.ds(p0, 16)]
            for h in range(H):
                accs[h] = accs[h] + plsc.load_gather(t_v, [hcasts[h], gi])
        for h in range(H):
            out_v[h, row, pl.ds(col, 16)] = accs[h] * invl
        return _

    lax.fori_loop(0, NPB, block, None)

    outs = [
        pltpu.async_copy(
            out_v.at[h], out_hbm.at[h, pl.ds(wid * RPW, RPW)], sem_out
        )
        for h in range(H)
    ]
    for d in outs:
        d.wait()


@functools.partial(jax.jit, static_argnums=())
def _sc_gather(lca_t, len_flat, t):
    mesh = plsc.VectorSubcoreMesh(core_axis_name="c", subcore_axis_name="s")
    f = functools.partial(
        pl.kernel,
        out_type=jax.ShapeDtypeStruct((H, N, N), jnp.float32),
        mesh=mesh,
        scratch_types=[
            pltpu.VMEM((L, PPW), jnp.int32),
            pltpu.VMEM((PPW,), jnp.int32),
            pltpu.VMEM((TW, ROWS), jnp.float32),
            pltpu.VMEM((H, RPW, N), jnp.float32),
            pltpu.SemaphoreType.DMA,
            pltpu.SemaphoreType.DMA,
        ],
        compiler_params=pltpu.CompilerParams(needs_layout_passes=False),
    )(_sc_body)
    return f(lca_t, len_flat, t)


def kernel(agent_lca, lca_len, emb_table, proj_w):
    lca_t = agent_lca.reshape(P, L).astype(jnp.int32).T
    len_flat = lca_len.reshape(-1).astype(jnp.int32)
    t = _project(emb_table, proj_w)
    return _sc_gather(lca_t, len_flat, t)
